# Initial kernel scaffold; baseline (speedup 1.0000x reference)
import jax
import jax.numpy as jnp
from jax.experimental import pallas as pl


def kernel(x_user, x_item, edge_u2i, edge_i2u, W_u2i, al_u2i, ar_u2i, b_u2i, W_i2u, al_i2u, ar_i2u, b_i2u, ln_g_user, ln_b_user, ln_g_item, ln_b_item):
    raise NotImplementedError("write your pallas kernel here")


# speedup vs baseline: 12.3557x; 12.3557x over previous
import functools

import jax
import jax.numpy as jnp
from jax import lax
from jax.experimental import pallas as pl
from jax.experimental.pallas import tpu as pltpu
from jax.experimental.pallas import tpu_sc as plsc

N = 10000
D = 128
HID = 128
H = 4
E = 160000

NPAD = 10240
NC = 2
NS = 16
NW = NC * NS
EPW = 5120
EPAD = NW * EPW

KA = 128
NSA = EPW // KA
BB = 2048
NBB = EPAD // BB
ROWS = NPAD // NW


def _mesh():
    return plsc.VectorSubcoreMesh(core_axis_name="c", subcore_axis_name="s",
                                  num_cores=NC, num_subcores=NS)


_SC_PARAMS = dict(compiler_params=pltpu.CompilerParams(
    needs_layout_passes=False))


def _proj_body(xu_ref, xi_ref, w1_ref, al1_ref, ar1_ref, w2_ref, al2_ref,
               ar2_ref, hsu_ref, hsi_ref, el1_ref, er1_ref, el2_ref, er2_ref):
    xu = xu_ref[...]
    xi = xi_ref[...]
    w1 = w1_ref[...]
    w2 = w2_ref[...]
    b = xu.shape[0]
    hu1 = jnp.dot(xu, w1, preferred_element_type=jnp.float32)
    hi2 = jnp.dot(xi, w2, preferred_element_type=jnp.float32)
    hi1 = jnp.dot(xi, w1, preferred_element_type=jnp.float32)
    hu2 = jnp.dot(xu, w2, preferred_element_type=jnp.float32)
    hsu_ref[...] = hu1
    hsi_ref[...] = hi2
    z = jnp.zeros((b, 16 - H), jnp.float32)

    def heads(hmat, avec):
        return jnp.concatenate(
            [jnp.sum(hmat.reshape(b, H, HID) * avec[...][None], axis=-1), z],
            axis=1)

    el1_ref[...] = heads(hu1, al1_ref)
    er1_ref[...] = heads(hi1, ar1_ref)
    el2_ref[...] = heads(hi2, al2_ref)
    er2_ref[...] = heads(hu2, ar2_ref)


def _proj(xu, xi, w1, al1, ar1, w2, al2, ar2):
    blk = 1024
    row_spec = pl.BlockSpec((blk, D), lambda i: (i, 0))
    full = lambda s: pl.BlockSpec(s, lambda i: tuple(0 for _ in s))
    hs_spec = pl.BlockSpec((blk, H * HID), lambda i: (i, 0))
    tab_spec = pl.BlockSpec((blk, 16), lambda i: (i, 0))
    out_shape = (
        jax.ShapeDtypeStruct((NPAD, H * HID), jnp.float32),
        jax.ShapeDtypeStruct((NPAD, H * HID), jnp.float32),
        jax.ShapeDtypeStruct((NPAD, 16), jnp.float32),
        jax.ShapeDtypeStruct((NPAD, 16), jnp.float32),
        jax.ShapeDtypeStruct((NPAD, 16), jnp.float32),
        jax.ShapeDtypeStruct((NPAD, 16), jnp.float32),
    )
    return pl.pallas_call(
        _proj_body,
        grid=(NPAD // blk,),
        in_specs=[row_spec, row_spec, full((D, H * HID)), full((H, HID)),
                  full((H, HID)), full((D, H * HID)), full((H, HID)),
                  full((H, HID))],
        out_specs=(hs_spec, hs_spec, tab_spec, tab_spec, tab_spec, tab_spec),
        out_shape=out_shape,
    )(xu, xi, w1, al1, ar1, w2, al2, ar2)


def _attn_body(el_tab, er_tab, s4_hbm, d4_hbm, a_out, el_v, er_v, s4_c, d4_c,
               ac_b):
    cid = lax.axis_index("c")
    sid = lax.axis_index("s")
    wid = sid * NC + cid
    pltpu.sync_copy(el_tab, el_v)
    pltpu.sync_copy(er_tab, er_v)

    def step(j, c):
        pltpu.sync_copy(s4_hbm.at[wid, j], s4_c)
        pltpu.sync_copy(d4_hbm.at[wid, j], d4_c)

        def grp(q, cc):
            si = s4_c[pl.ds(q * 16, 16)]
            di = d4_c[pl.ds(q * 16, 16)]
            x = plsc.load_gather(el_v, [si]) + plsc.load_gather(er_v, [di])
            ac_b[pl.ds(q * 16, 16)] = jnp.exp(jnp.maximum(x, 0.2 * x))
            return cc
        lax.fori_loop(0, KA * 4 // 16, grp, 0)
        pltpu.sync_copy(ac_b, a_out.at[pl.ds((wid * NSA + j) * KA * 4, KA * 4)])
        return c
    lax.fori_loop(0, NSA, step, 0)


@functools.lru_cache(maxsize=None)
def _attn_call_build():
    return pl.kernel(
        _attn_body,
        out_type=jax.ShapeDtypeStruct((EPAD * 4,), jnp.float32),
        mesh=_mesh(),
        scratch_types=[
            pltpu.VMEM((NPAD * 4,), jnp.float32),
            pltpu.VMEM((NPAD * 4,), jnp.float32),
            pltpu.VMEM((KA * 4,), jnp.int32),
            pltpu.VMEM((KA * 4,), jnp.int32),
            pltpu.VMEM((KA * 4,), jnp.float32),
        ],
        **_SC_PARAMS,
    )


def _dencomp_body(a_hbm, d4_hbm, den_out, den_t, ac_b, d4_c):
    cid = lax.axis_index("c")
    sid = lax.axis_index("s")
    wid = sid * NC + cid

    def zv(i, c):
        den_t[pl.ds(i * 16, 16)] = jnp.zeros((16,), jnp.float32)
        return c
    lax.fori_loop(0, NPAD * 4 // 16, zv, 0)

    lanes = lax.iota(jnp.int32, 16)
    rq = lanes // 4

    def step(j, c):
        pltpu.sync_copy(a_hbm.at[pl.ds((wid * NSA + j) * KA * 4, KA * 4)],
                        ac_b)
        pltpu.sync_copy(d4_hbm.at[wid, j], d4_c)

        def grp(q, cc):
            di = d4_c[pl.ds(q * 16, 16)]
            a = ac_b[pl.ds(q * 16, 16)]
            for m in range(4):
                plsc.addupdate_scatter(den_t, [di], a, mask=rq == m)
            return cc
        lax.fori_loop(0, KA * 4 // 16, grp, 0)
        return c
    lax.fori_loop(0, NSA, step, 0)
    pltpu.sync_copy(den_t, den_out.at[wid])


@functools.lru_cache(maxsize=None)
def _dencomp_call_build():
    return pl.kernel(
        _dencomp_body,
        out_type=jax.ShapeDtypeStruct((NW, NPAD * 4), jnp.float32),
        mesh=_mesh(),
        scratch_types=[
            pltpu.VMEM((NPAD * 4,), jnp.float32),
            pltpu.VMEM((KA * 4,), jnp.float32),
            pltpu.VMEM((KA * 4,), jnp.int32),
        ],
        **_SC_PARAMS,
    )


SLICE32 = NPAD * 4 // NW


def _denred_body(den_parts, den_c_out, acc_b, tmp_b):
    cid = lax.axis_index("c")
    sid = lax.axis_index("s")
    wid = sid * NC + cid
    base = wid * SLICE32
    pltpu.sync_copy(den_parts.at[0, pl.ds(base, SLICE32)], acc_b)
    for k in range(1, NW):
        pltpu.sync_copy(den_parts.at[k, pl.ds(base, SLICE32)], tmp_b)

        def add(g, c):
            s = pl.ds(g * 16, 16)
            acc_b[s] = acc_b[s] + tmp_b[s]
            return c
        lax.fori_loop(0, SLICE32 // 16, add, 0)
    pltpu.sync_copy(acc_b, den_c_out.at[pl.ds(base, SLICE32)])


@functools.lru_cache(maxsize=None)
def _denred_call_build():
    return pl.kernel(
        _denred_body,
        out_type=jax.ShapeDtypeStruct((NPAD * 4,), jnp.float32),
        mesh=_mesh(),
        scratch_types=[
            pltpu.VMEM((SLICE32,), jnp.float32),
            pltpu.VMEM((SLICE32,), jnp.float32),
        ],
        **_SC_PARAMS,
    )


def _norm_body(a_hbm, den_c, d4_hbm, att_out, den_v, ac_b, d4_c):
    cid = lax.axis_index("c")
    sid = lax.axis_index("s")
    wid = sid * NC + cid
    pltpu.sync_copy(den_c, den_v)

    def step(j, c):
        base = (wid * NSA + j) * KA * 4
        pltpu.sync_copy(a_hbm.at[pl.ds(base, KA * 4)], ac_b)
        pltpu.sync_copy(d4_hbm.at[wid, j], d4_c)

        def grp(q, cc):
            di = d4_c[pl.ds(q * 16, 16)]
            den = plsc.load_gather(den_v, [di])
            a = ac_b[pl.ds(q * 16, 16)]
            ac_b[pl.ds(q * 16, 16)] = a / ((den + 1e-9) * float(H))
            return cc
        lax.fori_loop(0, KA * 4 // 16, grp, 0)
        pltpu.sync_copy(ac_b, att_out.at[pl.ds(base, KA * 4)])
        return c
    lax.fori_loop(0, NSA, step, 0)


@functools.lru_cache(maxsize=None)
def _norm_call_build():
    return pl.kernel(
        _norm_body,
        out_type=jax.ShapeDtypeStruct((EPAD * 4,), jnp.float32),
        mesh=_mesh(),
        scratch_types=[
            pltpu.VMEM((NPAD * 4,), jnp.float32),
            pltpu.VMEM((KA * 4,), jnp.float32),
            pltpu.VMEM((KA * 4,), jnp.int32),
        ],
        **_SC_PARAMS,
    )


def _agg_body(hs_hbm, att_hbm, src_hbm, dst_hbm, out_hbm,
              out_l, src_b, dst_b, att_b, ceid, sidx, hs_b, sem1):
    cid = lax.axis_index("c")
    sid = lax.axis_index("s")
    wid = sid * NC + cid
    lo = wid * ROWS

    def zrow(i, c):
        for db in range(HID // 16):
            out_l[i, pl.ds(db * 16, 16)] = jnp.zeros((16,), jnp.float32)
        return c
    lax.fori_loop(0, ROWS, zrow, 0)

    lanes = lax.iota(jnp.int32, 16)

    def block(b, c):
        pltpu.sync_copy(src_hbm.at[pl.ds(b * BB, BB)], src_b)
        pltpu.sync_copy(dst_hbm.at[pl.ds(b * BB, BB)], dst_b)
        pltpu.sync_copy(att_hbm.at[pl.ds(b * BB * 4, BB * 4)], att_b)

        def scan(g, cur):
            dv = dst_b[pl.ds(g * 16, 16)]
            m = (dv >= lo) & (dv < lo + ROWS)
            plsc.store_compressed(ceid.at[pl.ds(cur, 16)], g * 16 + lanes,
                                  mask=m)
            return cur + plsc.all_reduce_population_count(m)[0]
        nsel = lax.fori_loop(0, BB // 16, scan, 0)

        def chunk(cc, c2):
            base = cc * 16
            leid = ceid[pl.ds(base, 16)]
            valid = (base + lanes) < nsel
            leid = jnp.where(valid, leid, 0)
            srcv = plsc.load_gather(src_b, [leid])
            dstv = plsc.load_gather(dst_b, [leid])
            dlv = jnp.clip(dstv - lo, 0, ROWS - 1)
            le4 = leid * 4
            zero = jnp.zeros((16,), jnp.float32)
            a0v = jnp.where(valid, plsc.load_gather(att_b, [le4]), zero)
            a1v = jnp.where(valid, plsc.load_gather(att_b, [le4 + 1]), zero)
            a2v = jnp.where(valid, plsc.load_gather(att_b, [le4 + 2]), zero)
            a3v = jnp.where(valid, plsc.load_gather(att_b, [le4 + 3]), zero)
            sidx[...] = srcv
            pltpu.async_copy(hs_hbm.at[sidx], hs_b, sem1).wait()
            for i in range(16):
                a0 = a0v[i]
                a1 = a1v[i]
                a2 = a2v[i]
                a3 = a3v[i]
                dl = dlv[i]
                for db in range(HID // 16):
                    o = db * 16
                    acc = a0 * hs_b[i, pl.ds(o, 16)]
                    acc = acc + a1 * hs_b[i, pl.ds(HID + o, 16)]
                    acc = acc + a2 * hs_b[i, pl.ds(2 * HID + o, 16)]
                    acc = acc + a3 * hs_b[i, pl.ds(3 * HID + o, 16)]
                    out_l[dl, pl.ds(o, 16)] = out_l[dl, pl.ds(o, 16)] + acc
            return c2
        lax.fori_loop(0, (nsel + 15) // 16, chunk, 0)
        return c
    lax.fori_loop(0, NBB, block, 0)

    pltpu.sync_copy(out_l, out_hbm.at[pl.ds(lo, ROWS)])


@functools.lru_cache(maxsize=None)
def _agg_call_build():
    return pl.kernel(
        _agg_body,
        out_type=jax.ShapeDtypeStruct((NPAD, HID), jnp.float32),
        mesh=_mesh(),
        scratch_types=[
            pltpu.VMEM((ROWS, HID), jnp.float32),
            pltpu.VMEM((BB,), jnp.int32),
            pltpu.VMEM((BB,), jnp.int32),
            pltpu.VMEM((BB * 4,), jnp.float32),
            pltpu.VMEM((BB + 16,), jnp.int32),
            pltpu.VMEM((16,), jnp.int32),
            pltpu.VMEM((16, H * HID), jnp.float32),
            pltpu.SemaphoreType.DMA,
        ],
        **_SC_PARAMS,
    )


def _post_body(conv_ref, x_ref, b_ref, g_ref, bb_ref, out_ref):
    bmean = jnp.mean(b_ref[...].reshape(H, HID), axis=0)
    y = conv_ref[...] + bmean[None, :] + x_ref[...]
    mu = jnp.mean(y, axis=-1, keepdims=True)
    yc = y - mu
    var = jnp.mean(yc * yc, axis=-1, keepdims=True)
    yn = yc * lax.rsqrt(var + 1e-5) * g_ref[...][None, :] + bb_ref[...][None, :]
    out_ref[...] = yn * 0.5 * (1.0 + lax.erf(yn * (2.0 ** -0.5)))


def _post(conv, x, b, g, bb):
    blk = 1024
    row_spec = pl.BlockSpec((blk, HID), lambda i: (i, 0))
    vec = lambda s: pl.BlockSpec(s, lambda i: tuple(0 for _ in s))
    return pl.pallas_call(
        _post_body,
        grid=(NPAD // blk,),
        in_specs=[row_spec, row_spec, vec((H * HID,)), vec((HID,)),
                  vec((HID,))],
        out_specs=row_spec,
        out_shape=jax.ShapeDtypeStruct((NPAD, HID), jnp.float32),
    )(conv, x, b, g, bb)


def _pad_edges(e):
    return jnp.concatenate([e.astype(jnp.int32),
                            jnp.full((EPAD - E,), N, jnp.int32)])


def _expand4(idx):
    return (idx[:, None] * 4 + jnp.arange(4, dtype=jnp.int32)).reshape(-1)


def _compact(tab):
    return tab[:, :H].reshape(-1)


def kernel(x_user, x_item, edge_u2i, edge_i2u, W_u2i, al_u2i, ar_u2i, b_u2i,
           W_i2u, al_i2u, ar_i2u, b_i2u, ln_g_user, ln_b_user, ln_g_item,
           ln_b_item):
    zpad = jnp.zeros((NPAD - N, D), jnp.float32)
    xu = jnp.concatenate([x_user, zpad], axis=0)
    xi = jnp.concatenate([x_item, zpad], axis=0)

    hs_u, hs_i, el1, er1, el2, er2 = _proj(
        xu, xi, W_u2i, al_u2i, ar_u2i, W_i2u, al_i2u, ar_i2u)

    attn_call = _attn_call_build()
    dencomp_call = _dencomp_call_build()
    denred_call = _denred_call_build()
    norm_call = _norm_call_build()
    agg_call = _agg_call_build()

    convs = []
    for edge, el, er, hs in ((edge_u2i, el1, er1, hs_u),
                             (edge_i2u, el2, er2, hs_i)):
        s = _pad_edges(edge[0])
        d = _pad_edges(edge[1])
        s4a = _expand4(s).reshape(NW, NSA, KA * 4)
        d4a = _expand4(d).reshape(NW, NSA, KA * 4)
        a_c = attn_call(_compact(el), _compact(er), s4a, d4a)
        den_parts = dencomp_call(a_c, d4a)
        den_c = denred_call(den_parts)
        att_c = norm_call(a_c, den_c, d4a)
        convs.append(agg_call(hs, att_c, s, d))
    conv_item, conv_user = convs

    out_item = _post(conv_item, xi, b_u2i, ln_g_item, ln_b_item)[:N]
    out_user = _post(conv_user, xu, b_i2u, ln_g_user, ln_b_user)[:N]
    return (out_user, out_item)


# speedup vs baseline: 12.4291x; 1.0059x over previous
import functools

import jax
import jax.numpy as jnp
from jax import lax
from jax.experimental import pallas as pl
from jax.experimental.pallas import tpu as pltpu
from jax.experimental.pallas import tpu_sc as plsc

N = 10000
D = 128
HID = 128
H = 4
E = 160000

NPAD = 10240
NC = 2
NS = 16
NW = NC * NS
EPW = 5120
EPAD = NW * EPW

KA = 128
NSA = EPW // KA
BB = 2048
NBB = EPAD // BB
ROWS = NPAD // NW


def _mesh():
    return plsc.VectorSubcoreMesh(core_axis_name="c", subcore_axis_name="s",
                                  num_cores=NC, num_subcores=NS)


_SC_PARAMS = dict(compiler_params=pltpu.CompilerParams(
    needs_layout_passes=False))


def _proj_body(xu_ref, xi_ref, w1_ref, al1_ref, ar1_ref, w2_ref, al2_ref,
               ar2_ref, hsu_ref, hsi_ref, el1_ref, er1_ref, el2_ref, er2_ref):
    xu = xu_ref[...]
    xi = xi_ref[...]
    w1 = w1_ref[...]
    w2 = w2_ref[...]
    b = xu.shape[0]
    hu1 = jnp.dot(xu, w1, preferred_element_type=jnp.float32)
    hi2 = jnp.dot(xi, w2, preferred_element_type=jnp.float32)
    hi1 = jnp.dot(xi, w1, preferred_element_type=jnp.float32)
    hu2 = jnp.dot(xu, w2, preferred_element_type=jnp.float32)
    hsu_ref[...] = hu1
    hsi_ref[...] = hi2
    z = jnp.zeros((b, 16 - H), jnp.float32)

    def heads(hmat, avec):
        return jnp.concatenate(
            [jnp.sum(hmat.reshape(b, H, HID) * avec[...][None], axis=-1), z],
            axis=1)

    el1_ref[...] = heads(hu1, al1_ref)
    er1_ref[...] = heads(hi1, ar1_ref)
    el2_ref[...] = heads(hi2, al2_ref)
    er2_ref[...] = heads(hu2, ar2_ref)


def _proj(xu, xi, w1, al1, ar1, w2, al2, ar2):
    blk = 1024
    row_spec = pl.BlockSpec((blk, D), lambda i: (i, 0))
    full = lambda s: pl.BlockSpec(s, lambda i: tuple(0 for _ in s))
    hs_spec = pl.BlockSpec((blk, H * HID), lambda i: (i, 0))
    tab_spec = pl.BlockSpec((blk, 16), lambda i: (i, 0))
    out_shape = (
        jax.ShapeDtypeStruct((NPAD, H * HID), jnp.float32),
        jax.ShapeDtypeStruct((NPAD, H * HID), jnp.float32),
        jax.ShapeDtypeStruct((NPAD, 16), jnp.float32),
        jax.ShapeDtypeStruct((NPAD, 16), jnp.float32),
        jax.ShapeDtypeStruct((NPAD, 16), jnp.float32),
        jax.ShapeDtypeStruct((NPAD, 16), jnp.float32),
    )
    return pl.pallas_call(
        _proj_body,
        grid=(NPAD // blk,),
        in_specs=[row_spec, row_spec, full((D, H * HID)), full((H, HID)),
                  full((H, HID)), full((D, H * HID)), full((H, HID)),
                  full((H, HID))],
        out_specs=(hs_spec, hs_spec, tab_spec, tab_spec, tab_spec, tab_spec),
        out_shape=out_shape,
    )(xu, xi, w1, al1, ar1, w2, al2, ar2)


def _attn_body(el_tab, er_tab, s4_hbm, d4_hbm, a_out, el_v, er_v, s4_c, d4_c,
               ac_b):
    cid = lax.axis_index("c")
    sid = lax.axis_index("s")
    wid = sid * NC + cid
    pltpu.sync_copy(el_tab, el_v)
    pltpu.sync_copy(er_tab, er_v)

    def step(j, c):
        pltpu.sync_copy(s4_hbm.at[wid, j], s4_c)
        pltpu.sync_copy(d4_hbm.at[wid, j], d4_c)

        def grp(q, cc):
            si = s4_c[pl.ds(q * 16, 16)]
            di = d4_c[pl.ds(q * 16, 16)]
            x = plsc.load_gather(el_v, [si]) + plsc.load_gather(er_v, [di])
            ac_b[pl.ds(q * 16, 16)] = jnp.exp(jnp.maximum(x, 0.2 * x))
            return cc
        lax.fori_loop(0, KA * 4 // 16, grp, 0)
        pltpu.sync_copy(ac_b, a_out.at[pl.ds((wid * NSA + j) * KA * 4, KA * 4)])
        return c
    lax.fori_loop(0, NSA, step, 0)


@functools.lru_cache(maxsize=None)
def _attn_call_build():
    return pl.kernel(
        _attn_body,
        out_type=jax.ShapeDtypeStruct((EPAD * 4,), jnp.float32),
        mesh=_mesh(),
        scratch_types=[
            pltpu.VMEM((NPAD * 4,), jnp.float32),
            pltpu.VMEM((NPAD * 4,), jnp.float32),
            pltpu.VMEM((KA * 4,), jnp.int32),
            pltpu.VMEM((KA * 4,), jnp.int32),
            pltpu.VMEM((KA * 4,), jnp.float32),
        ],
        **_SC_PARAMS,
    )


def _dencomp_body(a_hbm, d4_hbm, den_out, den_t, ac_b, d4_c):
    cid = lax.axis_index("c")
    sid = lax.axis_index("s")
    wid = sid * NC + cid

    def zv(i, c):
        den_t[pl.ds(i * 16, 16)] = jnp.zeros((16,), jnp.float32)
        return c
    lax.fori_loop(0, NPAD * 4 // 16, zv, 0)

    lanes = lax.iota(jnp.int32, 16)
    rq = lanes // 4

    def step(j, c):
        pltpu.sync_copy(a_hbm.at[pl.ds((wid * NSA + j) * KA * 4, KA * 4)],
                        ac_b)
        pltpu.sync_copy(d4_hbm.at[wid, j], d4_c)

        def grp(q, cc):
            di = d4_c[pl.ds(q * 16, 16)]
            a = ac_b[pl.ds(q * 16, 16)]
            for m in range(4):
                plsc.addupdate_scatter(den_t, [di], a, mask=rq == m)
            return cc
        lax.fori_loop(0, KA * 4 // 16, grp, 0)
        return c
    lax.fori_loop(0, NSA, step, 0)
    pltpu.sync_copy(den_t, den_out.at[wid])


@functools.lru_cache(maxsize=None)
def _dencomp_call_build():
    return pl.kernel(
        _dencomp_body,
        out_type=jax.ShapeDtypeStruct((NW, NPAD * 4), jnp.float32),
        mesh=_mesh(),
        scratch_types=[
            pltpu.VMEM((NPAD * 4,), jnp.float32),
            pltpu.VMEM((KA * 4,), jnp.float32),
            pltpu.VMEM((KA * 4,), jnp.int32),
        ],
        **_SC_PARAMS,
    )


SLICE32 = NPAD * 4 // NW


def _denred_body(den_parts, den_c_out, acc_b, tmp_b):
    cid = lax.axis_index("c")
    sid = lax.axis_index("s")
    wid = sid * NC + cid
    base = wid * SLICE32
    pltpu.sync_copy(den_parts.at[0, pl.ds(base, SLICE32)], acc_b)
    for k in range(1, NW):
        pltpu.sync_copy(den_parts.at[k, pl.ds(base, SLICE32)], tmp_b)

        def add(g, c):
            s = pl.ds(g * 16, 16)
            acc_b[s] = acc_b[s] + tmp_b[s]
            return c
        lax.fori_loop(0, SLICE32 // 16, add, 0)
    pltpu.sync_copy(acc_b, den_c_out.at[pl.ds(base, SLICE32)])


@functools.lru_cache(maxsize=None)
def _denred_call_build():
    return pl.kernel(
        _denred_body,
        out_type=jax.ShapeDtypeStruct((NPAD * 4,), jnp.float32),
        mesh=_mesh(),
        scratch_types=[
            pltpu.VMEM((SLICE32,), jnp.float32),
            pltpu.VMEM((SLICE32,), jnp.float32),
        ],
        **_SC_PARAMS,
    )


def _norm_body(a_hbm, den_c, d4_hbm, att_out, den_v, ac_b, d4_c):
    cid = lax.axis_index("c")
    sid = lax.axis_index("s")
    wid = sid * NC + cid
    pltpu.sync_copy(den_c, den_v)

    def step(j, c):
        base = (wid * NSA + j) * KA * 4
        pltpu.sync_copy(a_hbm.at[pl.ds(base, KA * 4)], ac_b)
        pltpu.sync_copy(d4_hbm.at[wid, j], d4_c)

        def grp(q, cc):
            di = d4_c[pl.ds(q * 16, 16)]
            den = plsc.load_gather(den_v, [di])
            a = ac_b[pl.ds(q * 16, 16)]
            ac_b[pl.ds(q * 16, 16)] = a / ((den + 1e-9) * float(H))
            return cc
        lax.fori_loop(0, KA * 4 // 16, grp, 0)
        pltpu.sync_copy(ac_b, att_out.at[pl.ds(base, KA * 4)])
        return c
    lax.fori_loop(0, NSA, step, 0)


@functools.lru_cache(maxsize=None)
def _norm_call_build():
    return pl.kernel(
        _norm_body,
        out_type=jax.ShapeDtypeStruct((EPAD * 4,), jnp.float32),
        mesh=_mesh(),
        scratch_types=[
            pltpu.VMEM((NPAD * 4,), jnp.float32),
            pltpu.VMEM((KA * 4,), jnp.float32),
            pltpu.VMEM((KA * 4,), jnp.int32),
        ],
        **_SC_PARAMS,
    )


def _agg_body(hs_hbm, att_hbm, src_hbm, dst_hbm, out_hbm,
              out_l, src_b, dst_b, att_b, ceid, sidx, hs_b, sem1):
    cid = lax.axis_index("c")
    sid = lax.axis_index("s")
    wid = sid * NC + cid
    lo = wid * ROWS

    def zrow(i, c):
        for db in range(HID // 16):
            out_l[i, pl.ds(db * 16, 16)] = jnp.zeros((16,), jnp.float32)
        return c
    lax.fori_loop(0, ROWS, zrow, 0)

    lanes = lax.iota(jnp.int32, 16)

    def block(b, c):
        pltpu.sync_copy(src_hbm.at[pl.ds(b * BB, BB)], src_b)
        pltpu.sync_copy(dst_hbm.at[pl.ds(b * BB, BB)], dst_b)
        pltpu.sync_copy(att_hbm.at[pl.ds(b * BB * 4, BB * 4)], att_b)

        def scan(g, cur):
            dv = dst_b[pl.ds(g * 16, 16)]
            m = (dv >= lo) & (dv < lo + ROWS)
            plsc.store_compressed(ceid.at[pl.ds(cur, 16)], g * 16 + lanes,
                                  mask=m)
            return cur + plsc.all_reduce_population_count(m)[0]
        nsel = lax.fori_loop(0, BB // 16, scan, 0)

        nc = (nsel + 15) // 16

        def issue(cc):
            par = cc % 2
            leid = ceid[pl.ds(cc * 16, 16)]
            leid = jnp.where((cc * 16 + lanes) < nsel, leid, 0)
            sidx[par] = plsc.load_gather(src_b, [leid])
            pltpu.async_copy(hs_hbm.at[sidx.at[par]], hs_b.at[par],
                             sem1.at[par])

        @pl.when(nc > 0)
        def _():
            issue(0)

        def chunk(cc, c2):
            par = cc % 2

            @pl.when(cc + 1 < nc)
            def _():
                issue(cc + 1)
            base = cc * 16
            leid = ceid[pl.ds(base, 16)]
            valid = (base + lanes) < nsel
            leid = jnp.where(valid, leid, 0)
            dstv = plsc.load_gather(dst_b, [leid])
            dlv = jnp.clip(dstv - lo, 0, ROWS - 1)
            le4 = leid * 4
            zero = jnp.zeros((16,), jnp.float32)
            a0v = jnp.where(valid, plsc.load_gather(att_b, [le4]), zero)
            a1v = jnp.where(valid, plsc.load_gather(att_b, [le4 + 1]), zero)
            a2v = jnp.where(valid, plsc.load_gather(att_b, [le4 + 2]), zero)
            a3v = jnp.where(valid, plsc.load_gather(att_b, [le4 + 3]), zero)
            pltpu.make_async_copy(hs_hbm.at[sidx.at[par]], hs_b.at[par],
                                  sem1.at[par]).wait()
            for i in range(16):
                a0 = a0v[i]
                a1 = a1v[i]
                a2 = a2v[i]
                a3 = a3v[i]
                dl = dlv[i]
                for db in range(HID // 16):
                    o = db * 16
                    acc = a0 * hs_b[par, i, pl.ds(o, 16)]
                    acc = acc + a1 * hs_b[par, i, pl.ds(HID + o, 16)]
                    acc = acc + a2 * hs_b[par, i, pl.ds(2 * HID + o, 16)]
                    acc = acc + a3 * hs_b[par, i, pl.ds(3 * HID + o, 16)]
                    out_l[dl, pl.ds(o, 16)] = out_l[dl, pl.ds(o, 16)] + acc
            return c2
        lax.fori_loop(0, nc, chunk, 0)
        return c
    lax.fori_loop(0, NBB, block, 0)

    pltpu.sync_copy(out_l, out_hbm.at[pl.ds(lo, ROWS)])


@functools.lru_cache(maxsize=None)
def _agg_call_build():
    return pl.kernel(
        _agg_body,
        out_type=jax.ShapeDtypeStruct((NPAD, HID), jnp.float32),
        mesh=_mesh(),
        scratch_types=[
            pltpu.VMEM((ROWS, HID), jnp.float32),
            pltpu.VMEM((BB,), jnp.int32),
            pltpu.VMEM((BB,), jnp.int32),
            pltpu.VMEM((BB * 4,), jnp.float32),
            pltpu.VMEM((BB + 16,), jnp.int32),
            pltpu.VMEM((2, 16), jnp.int32),
            pltpu.VMEM((2, 16, H * HID), jnp.float32),
            pltpu.SemaphoreType.DMA((2,)),
        ],
        **_SC_PARAMS,
    )


def _post_body(conv_ref, x_ref, b_ref, g_ref, bb_ref, out_ref):
    bmean = jnp.mean(b_ref[...].reshape(H, HID), axis=0)
    y = conv_ref[...] + bmean[None, :] + x_ref[...]
    mu = jnp.mean(y, axis=-1, keepdims=True)
    yc = y - mu
    var = jnp.mean(yc * yc, axis=-1, keepdims=True)
    yn = yc * lax.rsqrt(var + 1e-5) * g_ref[...][None, :] + bb_ref[...][None, :]
    out_ref[...] = yn * 0.5 * (1.0 + lax.erf(yn * (2.0 ** -0.5)))


def _post(conv, x, b, g, bb):
    blk = 1024
    row_spec = pl.BlockSpec((blk, HID), lambda i: (i, 0))
    vec = lambda s: pl.BlockSpec(s, lambda i: tuple(0 for _ in s))
    return pl.pallas_call(
        _post_body,
        grid=(NPAD // blk,),
        in_specs=[row_spec, row_spec, vec((H * HID,)), vec((HID,)),
                  vec((HID,))],
        out_specs=row_spec,
        out_shape=jax.ShapeDtypeStruct((NPAD, HID), jnp.float32),
    )(conv, x, b, g, bb)


def _pad_edges(e):
    return jnp.concatenate([e.astype(jnp.int32),
                            jnp.full((EPAD - E,), N, jnp.int32)])


def _expand4(idx):
    return (idx[:, None] * 4 + jnp.arange(4, dtype=jnp.int32)).reshape(-1)


def _compact(tab):
    return tab[:, :H].reshape(-1)


def kernel(x_user, x_item, edge_u2i, edge_i2u, W_u2i, al_u2i, ar_u2i, b_u2i,
           W_i2u, al_i2u, ar_i2u, b_i2u, ln_g_user, ln_b_user, ln_g_item,
           ln_b_item):
    zpad = jnp.zeros((NPAD - N, D), jnp.float32)
    xu = jnp.concatenate([x_user, zpad], axis=0)
    xi = jnp.concatenate([x_item, zpad], axis=0)

    hs_u, hs_i, el1, er1, el2, er2 = _proj(
        xu, xi, W_u2i, al_u2i, ar_u2i, W_i2u, al_i2u, ar_i2u)

    attn_call = _attn_call_build()
    dencomp_call = _dencomp_call_build()
    denred_call = _denred_call_build()
    norm_call = _norm_call_build()
    agg_call = _agg_call_build()

    convs = []
    for edge, el, er, hs in ((edge_u2i, el1, er1, hs_u),
                             (edge_i2u, el2, er2, hs_i)):
        s = _pad_edges(edge[0])
        d = _pad_edges(edge[1])
        s4a = _expand4(s).reshape(NW, NSA, KA * 4)
        d4a = _expand4(d).reshape(NW, NSA, KA * 4)
        a_c = attn_call(_compact(el), _compact(er), s4a, d4a)
        den_parts = dencomp_call(a_c, d4a)
        den_c = denred_call(den_parts)
        att_c = norm_call(a_c, den_c, d4a)
        convs.append(agg_call(hs, att_c, s, d))
    conv_item, conv_user = convs

    out_item = _post(conv_item, xi, b_u2i, ln_g_item, ln_b_item)[:N]
    out_user = _post(conv_user, xu, b_i2u, ln_g_user, ln_b_user)[:N]
    return (out_user, out_item)
